# baseline (speedup 1.0000x reference)
import jax
import jax.numpy as jnp
from jax.experimental import pallas as pl

D_OUT = 4096
D_IN = 4096
BLOCK = 64
N_BLOCKS = D_IN // BLOCK

ROW_BLK = 256


def _body(m_ref, s_ref, o_ref):
    x = m_ref[...]
    s = s_ref[...]
    s_safe = jnp.where(s == 0.0, 1.0, s)
    r75 = 7.5 / s_safe
    m = s * (2.0 / 15.0)
    u = x * r75 + 8.0
    cnt = jnp.floor(u)
    o_ref[...] = (cnt - 7.5) * m


def kernel(master, scale, centroids):
    del centroids
    m3 = master.reshape(D_OUT, N_BLOCKS, BLOCK)
    s3 = scale.reshape(D_OUT, N_BLOCKS, 1)
    grid = (D_OUT // ROW_BLK,)
    out = pl.pallas_call(
        _body,
        grid=grid,
        in_specs=[
            pl.BlockSpec((ROW_BLK, N_BLOCKS, BLOCK), lambda i: (i, 0, 0)),
            pl.BlockSpec((ROW_BLK, N_BLOCKS, 1), lambda i: (i, 0, 0)),
        ],
        out_specs=pl.BlockSpec((ROW_BLK, N_BLOCKS, BLOCK), lambda i: (i, 0, 0)),
        out_shape=jax.ShapeDtypeStruct((D_OUT, N_BLOCKS, BLOCK), jnp.float32),
    )(m3, s3)
    return out.reshape(D_OUT, D_IN)


# speedup vs baseline: 4.3680x; 4.3680x over previous
import jax
import jax.numpy as jnp
from jax.experimental import pallas as pl

D_OUT = 4096
D_IN = 4096
BLOCK = 64
N_BLOCKS = D_IN // BLOCK

ROW_BLK = 256


def _body(m_ref, s_ref, o_ref):
    row = jax.lax.broadcasted_iota(jnp.int32, (N_BLOCKS, D_IN), 0)
    col = jax.lax.broadcasted_iota(jnp.int32, (N_BLOCKS, D_IN), 1)
    e = (row == col // BLOCK).astype(jnp.float32)

    s = s_ref[...]
    s_safe = jnp.where(s == 0.0, 1.0, s)
    r75 = 7.5 / s_safe
    m = s * (2.0 / 15.0)
    hi = jax.lax.Precision.HIGHEST
    r75e = jnp.dot(r75, e, precision=hi)
    me = jnp.dot(m, e, precision=hi)

    x = m_ref[...]
    cnt = jnp.floor(x * r75e + 8.0)
    o_ref[...] = (cnt - 7.5) * me


def kernel(master, scale, centroids):
    del centroids
    grid = (D_OUT // ROW_BLK,)
    return pl.pallas_call(
        _body,
        grid=grid,
        in_specs=[
            pl.BlockSpec((ROW_BLK, D_IN), lambda i: (i, 0)),
            pl.BlockSpec((ROW_BLK, N_BLOCKS), lambda i: (i, 0)),
        ],
        out_specs=pl.BlockSpec((ROW_BLK, D_IN), lambda i: (i, 0)),
        out_shape=jax.ShapeDtypeStruct((D_OUT, D_IN), jnp.float32),
    )(master, scale)
